# baseline (speedup 1.0000x reference)
import functools

import jax
import jax.numpy as jnp
from jax import lax
from jax.experimental import pallas as pl
from jax.experimental.pallas import tpu as pltpu
from jax.experimental.pallas import tpu_sc as plsc

_NC = 2
_NS = 16
_NW = _NC * _NS
_LANES = 16

_CHUNK = 64
_NCHUNKS = 160
_PER_W = _CHUNK * _NCHUNKS
_GROUPS = _CHUNK // _LANES


def _matmul_tc(x, w):
    n, d_in = x.shape
    d_out = w.shape[1]
    blk = 1000
    assert n % blk == 0

    def body(x_ref, w_ref, o_ref):
        o_ref[...] = jnp.dot(x_ref[...].astype(jnp.bfloat16),
                             w_ref[...].astype(jnp.bfloat16),
                             preferred_element_type=jnp.float32)

    return pl.pallas_call(
        body,
        grid=(n // blk,),
        in_specs=[
            pl.BlockSpec((blk, d_in), lambda i: (i, 0)),
            pl.BlockSpec((d_in, d_out), lambda i: (0, 0)),
        ],
        out_specs=pl.BlockSpec((blk, d_out), lambda i: (i, 0)),
        out_shape=jax.ShapeDtypeStruct((n, d_out), jnp.float32),
    )(x, w)


def _edge_weights_sc(h, comb, a_bcast):
    n, d = h.shape
    assert d == 128
    n_stagers = 10
    assert n % n_stagers == 0
    rows_per_tile = n // n_stagers
    assert rows_per_tile % 8 == 0
    e_pad = _NW * _PER_W

    mesh = plsc.VectorSubcoreMesh(core_axis_name="c", subcore_axis_name="s")

    @functools.partial(
        pl.kernel,
        mesh=mesh,
        compiler_params=pltpu.CompilerParams(needs_layout_passes=False),
        out_type=jax.ShapeDtypeStruct((e_pad,), jnp.float32),
        scratch_types=[
            pltpu.VMEM((2, 2 * _CHUNK), jnp.int32),
            pltpu.VMEM((2 * _CHUNK, 128), jnp.float32),
            pltpu.VMEM((2 * _CHUNK, 128), jnp.float32),
            pltpu.VMEM((2, _CHUNK), jnp.float32),
            pltpu.VMEM((128, _LANES), jnp.float32),
            pltpu.VMEM_SHARED((n, 128), jnp.float32),
            pltpu.SemaphoreType.DMA,
            pltpu.SemaphoreType.DMA,
            pltpu.SemaphoreType.DMA,
            pltpu.SemaphoreType.DMA,
            pltpu.SemaphoreType.DMA,
            pltpu.SemaphoreType.DMA,
        ],
    )
    def edge_kernel(h_hbm, comb_hbm, a_hbm, out_hbm,
                    cidx, crows0, crows1,
                    obuf, a_v, h_sh,
                    sr0, sr1, si0, si1, so0, so1):
        sid = lax.axis_index("s")
        wid = sid * _NC + lax.axis_index("c")
        base = wid * _PER_W

        @pl.when(sid < n_stagers)
        def stage():
            row0 = sid * rows_per_tile
            pltpu.sync_copy(h_hbm.at[pl.ds(row0, rows_per_tile)],
                            h_sh.at[pl.ds(row0, rows_per_tile)])
        pltpu.sync_copy(a_hbm, a_v)
        plsc.subcore_barrier()

        rbufs = ((crows0, sr0), (crows1, sr1))
        isems = (si0, si1)
        osems = (so0, so1)
        eids = [lax.iota(jnp.int32, _LANES) + g * _LANES
                for g in range(_GROUPS)]
        iot = lax.iota(jnp.int32, _LANES)

        def idx_copy(cc, b):
            lo = 2 * (base + cc * _CHUNK)
            c1 = pltpu.make_async_copy(
                comb_hbm.at[pl.ds(lo, 2 * _CHUNK)], cidx.at[b], isems[b])
            return (c1,)

        def gather_copy(b):
            crows, sr = rbufs[b]
            c1 = pltpu.make_async_copy(h_sh.at[cidx.at[b]], crows, sr)
            return (c1,)

        def out_copy(cc, b):
            lo = base + cc * _CHUNK
            return pltpu.make_async_copy(
                obuf.at[b], out_hbm.at[pl.ds(lo, _CHUNK)], osems[b])

        for cps in (idx_copy(0, 0), idx_copy(1, 1)):
            for c in cps:
                c.start()
        for c in idx_copy(0, 0):
            c.wait()
        for c in gather_copy(0):
            c.start()

        @pl.loop(0, _NCHUNKS, step=2)
        def chunk_loop(c):
            for b in range(2):
                cc = c + b

                @pl.when(cc + 1 < _NCHUNKS)
                def start_next():
                    for cp in idx_copy(cc + 1, 1 - b):
                        cp.wait()
                    for cp in gather_copy(1 - b):
                        cp.start()

                for cp in gather_copy(b):
                    cp.wait()

                def kbody(kk, accs):
                    kv = (iot + kk) & 127
                    ak = a_v[kk]
                    new = []
                    crows = rbufs[b][0]
                    for g in range(_GROUPS):
                        vs = plsc.load_gather(crows, [eids[g], kv])
                        vd = plsc.load_gather(crows, [eids[g] + _CHUNK, kv])
                        di = jnp.abs(vs - vd)
                        u = plsc.bitcast(di, jnp.int32)
                        db = plsc.bitcast(
                            (u + 0x8000) & jnp.int32(-65536), jnp.float32)
                        new.append(accs[g] + db * ak)
                    return tuple(new)

                @pl.when(cc >= 2)
                def drain_out():
                    out_copy(cc - 2, b).wait()

                accs = lax.fori_loop(
                    0, 128, kbody,
                    tuple(jnp.zeros((_LANES,), jnp.float32)
                          for _ in range(_GROUPS)),
                    unroll=4)
                for g in range(_GROUPS):
                    obuf[b, pl.ds(g * _LANES, _LANES)] = (
                        jnp.maximum(accs[g], 0.0))
                out_copy(cc, b).start()

                @pl.when(cc + 2 < _NCHUNKS)
                def refill_idx():
                    for cp in idx_copy(cc + 2, b):
                        cp.start()

        out_copy(_NCHUNKS - 2, 0).wait()
        out_copy(_NCHUNKS - 1, 1).wait()

    return edge_kernel(h, comb, a_bcast)


def kernel(inputs, edge, weight, a):
    h = _matmul_tc(inputs, weight)
    e = edge.shape[1]
    e_pad = _NW * _PER_W
    edge_i = jnp.asarray(edge, jnp.int32)
    edge_p = jnp.pad(edge_i, ((0, 0), (0, e_pad - e)))
    comb = jnp.transpose(
        edge_p.reshape(2, _NW, _NCHUNKS, _CHUNK),
        (1, 2, 0, 3)).reshape(-1)
    au = lax.bitcast_convert_type(
        a.reshape(-1).astype(jnp.float32), jnp.int32)
    ar = (au + 0x7FFF + ((au >> 16) & 1)) & jnp.int32(-65536)
    a_rounded = lax.bitcast_convert_type(ar, jnp.float32)
    rot = (jnp.arange(128)[:, None] + jnp.arange(_LANES)[None, :]) % 128
    a_bcast = a_rounded[rot]
    ew_pad = _edge_weights_sc(h, comb, a_bcast)
    return (h, ew_pad[:e])


# speedup vs baseline: 1.1525x; 1.1525x over previous
import functools

import jax
import jax.numpy as jnp
from jax import lax
from jax.experimental import pallas as pl
from jax.experimental.pallas import tpu as pltpu
from jax.experimental.pallas import tpu_sc as plsc

_NC = 2
_NS = 16
_NW = _NC * _NS
_LANES = 16

_CHUNK = 64
_NCHUNKS = 160
_PER_W = _CHUNK * _NCHUNKS
_GROUPS = _CHUNK // _LANES


def _matmul_tc(x, w):
    n, d_in = x.shape
    d_out = w.shape[1]
    blk = 1000
    assert n % blk == 0

    def body(x_ref, w_ref, o_ref):
        o_ref[...] = jnp.dot(x_ref[...].astype(jnp.bfloat16),
                             w_ref[...].astype(jnp.bfloat16),
                             preferred_element_type=jnp.float32)

    return pl.pallas_call(
        body,
        grid=(n // blk,),
        in_specs=[
            pl.BlockSpec((blk, d_in), lambda i: (i, 0)),
            pl.BlockSpec((d_in, d_out), lambda i: (0, 0)),
        ],
        out_specs=pl.BlockSpec((blk, d_out), lambda i: (i, 0)),
        out_shape=jax.ShapeDtypeStruct((n, d_out), jnp.float32),
    )(x, w)


def _edge_weights_sc(h, src_3d, dst_3d, a_bcast):
    n, d = h.shape
    assert d == 128
    n_stagers = 10
    assert n % n_stagers == 0
    rows_per_tile = n // n_stagers
    assert rows_per_tile % 8 == 0
    e_pad = _NW * _PER_W

    mesh = plsc.VectorSubcoreMesh(core_axis_name="c", subcore_axis_name="s")

    @functools.partial(
        pl.kernel,
        mesh=mesh,
        compiler_params=pltpu.CompilerParams(needs_layout_passes=False),
        out_type=jax.ShapeDtypeStruct((e_pad,), jnp.float32),
        scratch_types=[
            pltpu.VMEM((2, _CHUNK), jnp.int32),
            pltpu.VMEM((2, _CHUNK), jnp.int32),
            pltpu.VMEM((_CHUNK, 128), jnp.float32),
            pltpu.VMEM((_CHUNK, 128), jnp.float32),
            pltpu.VMEM((_CHUNK, 128), jnp.float32),
            pltpu.VMEM((_CHUNK, 128), jnp.float32),
            pltpu.VMEM((2, _CHUNK), jnp.float32),
            pltpu.VMEM((128, _LANES), jnp.float32),
            pltpu.VMEM_SHARED((n, 128), jnp.float32),
            pltpu.SemaphoreType.DMA,
            pltpu.SemaphoreType.DMA,
            pltpu.SemaphoreType.DMA,
            pltpu.SemaphoreType.DMA,
            pltpu.SemaphoreType.DMA,
            pltpu.SemaphoreType.DMA,
            pltpu.SemaphoreType.DMA,
            pltpu.SemaphoreType.DMA,
        ],
    )
    def edge_kernel(h_hbm, src_hbm, dst_hbm, a_hbm, out_hbm,
                    sidx, didx, srows0, drows0, srows1, drows1,
                    obuf, a_v, h_sh,
                    ss0, sd0, ss1, sd1, si0, si1, so0, so1):
        sid = lax.axis_index("s")
        wid = sid * _NC + lax.axis_index("c")
        base = wid * _PER_W

        @pl.when(sid < n_stagers)
        def stage():
            row0 = sid * rows_per_tile
            pltpu.sync_copy(h_hbm.at[pl.ds(row0, rows_per_tile)],
                            h_sh.at[pl.ds(row0, rows_per_tile)])
        pltpu.sync_copy(a_hbm, a_v)
        plsc.subcore_barrier()

        rbufs = ((srows0, drows0, ss0, sd0), (srows1, drows1, ss1, sd1))
        isems = (si0, si1)
        osems = (so0, so1)
        eids = [lax.iota(jnp.int32, _LANES) + g * _LANES
                for g in range(_GROUPS)]
        iot = lax.iota(jnp.int32, _LANES)

        def idx_copy(cc, b):
            lo = base + cc * _CHUNK
            c1 = pltpu.make_async_copy(
                src_hbm.at[pl.ds(lo, _CHUNK)], sidx.at[b], isems[b])
            c2 = pltpu.make_async_copy(
                dst_hbm.at[pl.ds(lo, _CHUNK)], didx.at[b], isems[b])
            return c1, c2

        def gather_copy(b):
            srows, drows, ss, sd = rbufs[b]
            c1 = pltpu.make_async_copy(h_sh.at[sidx.at[b]], srows, ss)
            c2 = pltpu.make_async_copy(h_sh.at[didx.at[b]], drows, sd)
            return c1, c2

        def out_copy(cc, b):
            lo = base + cc * _CHUNK
            return pltpu.make_async_copy(
                obuf.at[b], out_hbm.at[pl.ds(lo, _CHUNK)], osems[b])

        for c1, c2 in (idx_copy(0, 0), idx_copy(1, 1)):
            c1.start()
            c2.start()
        for c in idx_copy(0, 0):
            c.wait()
        for c in gather_copy(0):
            c.start()

        @pl.loop(0, _NCHUNKS, step=2)
        def chunk_loop(c):
            for b in range(2):
                cc = c + b

                @pl.when(cc + 1 < _NCHUNKS)
                def start_next():
                    for cp in idx_copy(cc + 1, 1 - b):
                        cp.wait()
                    for cp in gather_copy(1 - b):
                        cp.start()

                for cp in gather_copy(b):
                    cp.wait()

                def kbody(kk, accs):
                    kv = (iot + kk) & 127
                    ak = a_v[kk]
                    new = []
                    srows, drows = rbufs[b][0], rbufs[b][1]
                    for g in range(_GROUPS):
                        vs = plsc.load_gather(srows, [eids[g], kv])
                        vd = plsc.load_gather(drows, [eids[g], kv])
                        di = jnp.abs(vs - vd)
                        u = plsc.bitcast(di, jnp.int32)
                        db = plsc.bitcast(
                            (u + 0x8000) & jnp.int32(-65536), jnp.float32)
                        new.append(accs[g] + db * ak)
                    return tuple(new)

                @pl.when(cc >= 2)
                def drain_out():
                    out_copy(cc - 2, b).wait()

                accs = lax.fori_loop(
                    0, 128, kbody,
                    tuple(jnp.zeros((_LANES,), jnp.float32)
                          for _ in range(_GROUPS)),
                    unroll=4)
                for g in range(_GROUPS):
                    obuf[b, pl.ds(g * _LANES, _LANES)] = (
                        jnp.maximum(accs[g], 0.0))
                out_copy(cc, b).start()

                @pl.when(cc + 2 < _NCHUNKS)
                def refill_idx():
                    for cp in idx_copy(cc + 2, b):
                        cp.start()

        out_copy(_NCHUNKS - 2, 0).wait()
        out_copy(_NCHUNKS - 1, 1).wait()

    return edge_kernel(h, src_3d, dst_3d, a_bcast)


def kernel(inputs, edge, weight, a):
    h = _matmul_tc(inputs, weight)
    e = edge.shape[1]
    e_pad = _NW * _PER_W
    edge_i = jnp.asarray(edge, jnp.int32)
    edge_p = jnp.pad(edge_i, ((0, 0), (0, e_pad - e)))
    src_3d = edge_p[0]
    dst_3d = edge_p[1]
    au = lax.bitcast_convert_type(
        a.reshape(-1).astype(jnp.float32), jnp.int32)
    ar = (au + 0x7FFF + ((au >> 16) & 1)) & jnp.int32(-65536)
    a_rounded = lax.bitcast_convert_type(ar, jnp.float32)
    rot = (jnp.arange(128)[:, None] + jnp.arange(_LANES)[None, :]) % 128
    a_bcast = a_rounded[rot]
    ew_pad = _edge_weights_sc(h, src_3d, dst_3d, a_bcast)
    return (h, ew_pad[:e])
